# baseline (speedup 1.0000x reference)
import functools

import jax
import jax.numpy as jnp
from jax import lax
from jax.experimental import pallas as pl
from jax.experimental.pallas import tpu as pltpu
from jax.experimental.pallas import tpu_sc as plsc

N = 10000
E = 320000
D = 128
DE = 16
H = 128
G = 64

NC = 2
NS = 16
NW = NC * NS
EPW = E // NW
K = 80
NCHUNK = EPW // K
NP_ = 10240
RPT = NP_ // NS
ZR = 128

_SC_MESH = plsc.VectorSubcoreMesh(core_axis_name="c", subcore_axis_name="s")


NPAIR = (NCHUNK - 1) // 2


def _sc_scat_body(row_hbm, col_hbm, ew_hbm, z_hbm, out_hbm,
                  ridx_f, cidx_f, cidx0, cidx1, ewv0, ewv1, rows0, rows1, acc,
                  sem0, sem1, ssem0, ssem1):
    cid = lax.axis_index("c")
    sid = lax.axis_index("s")
    wid = sid * NC + cid
    ebase = wid * EPW

    pltpu.sync_copy(row_hbm.at[pl.ds(ebase, EPW)], ridx_f)
    pltpu.sync_copy(col_hbm.at[pl.ds(ebase, EPW)], cidx_f)

    @pl.loop(0, K)
    def _zero(i):
        for j in range(D // 16):
            rows0[i, pl.ds(j * 16, 16)] = jnp.zeros((16,), jnp.float32)

    for kk in range(RPT // K):
        pltpu.sync_copy(rows0, acc.at[pl.ds(sid * RPT + kk * K, K)])
    plsc.subcore_barrier()

    def gather_issue(g, rows, sem):
        pltpu.async_copy(z_hbm.at[ridx_f.at[pl.ds(g * K, K)]], rows, sem)

    def gather_wait(g, rows, sem):
        pltpu.make_async_copy(z_hbm.at[ridx_f.at[pl.ds(g * K, K)]], rows,
                              sem).wait()

    def ew_issue(g, buf, sem):
        pltpu.async_copy(ew_hbm.at[pl.ds(ebase + g * K, K)], buf, sem)

    def ew_wait(g, buf, sem):
        pltpu.make_async_copy(ew_hbm.at[pl.ds(ebase + g * K, K)], buf,
                              sem).wait()

    def scale(g, rows, ewv):
        @pl.loop(0, K // 16)
        def _scale(e16):
            ewvec = ewv[pl.ds(e16 * 16, 16)]
            for l in range(16):
                e = e16 * 16 + l
                s = ewvec[l]
                for j in range(D // 16):
                    sl = pl.ds(j * 16, 16)
                    rows[e, sl] = rows[e, sl] * s

    def cidx_stage(g, cidx_cur):
        for i in range(K // 16):
            cidx_cur[pl.ds(i * 16, 16)] = cidx_f[pl.ds(g * K + i * 16, 16)]

    def scat_issue(rows, cidx_cur, ssem):
        pltpu.async_copy(rows, acc.at[cidx_cur], ssem, add=True)

    def scat_wait(rows, cidx_cur, ssem):
        pltpu.make_async_copy(rows, acc.at[cidx_cur], ssem).wait()

    s0 = (rows0, cidx0, ewv0, sem0, ssem0)
    s1 = (rows1, cidx1, ewv1, sem1, ssem1)

    def process(c, slot, other, first=False, last=False):
        rows, cidx, ewv, gsem, ssem = slot
        orows, ocidx, oewv, ogsem, ossem = other
        gather_wait(c, rows, gsem)
        ew_wait(c, ewv, gsem)
        scale(c, rows, ewv)
        cidx_stage(c, cidx)
        if not first:
            scat_wait(orows, ocidx, ossem)
        if not last:
            gather_issue(c + 1, orows, ogsem)
            ew_issue(c + 1, oewv, ogsem)
        scat_issue(rows, cidx, ssem)

    gather_issue(0, rows0, sem0)
    ew_issue(0, ewv0, sem0)
    process(0, s0, s1, first=True)

    @pl.loop(0, (NCHUNK - 3) // 2)
    def _pair(t):
        ga = 2 * t + 1
        process(ga, s1, s0)
        process(ga + 1, s0, s1)

    process(NCHUNK - 2, s1, s0)
    process(NCHUNK - 1, s0, s1, last=True)
    scat_wait(rows0, cidx0, ssem0)

    plsc.subcore_barrier()
    for kk in range(RPT // ZR):
        r0 = sid * RPT + kk * ZR
        pltpu.sync_copy(acc.at[pl.ds(r0, ZR)], out_hbm.at[cid, pl.ds(r0, ZR)])


_sc_scatter = functools.partial(
    pl.kernel,
    out_type=jax.ShapeDtypeStruct((NC, NP_, D), jnp.float32),
    mesh=_SC_MESH,
    scratch_types=[
        pltpu.VMEM((EPW,), jnp.int32),
        pltpu.VMEM((EPW,), jnp.int32),
        pltpu.VMEM((K,), jnp.int32),
        pltpu.VMEM((K,), jnp.int32),
        pltpu.VMEM((K,), jnp.float32),
        pltpu.VMEM((K,), jnp.float32),
        pltpu.VMEM((K, D), jnp.float32),
        pltpu.VMEM((K, D), jnp.float32),
        pltpu.VMEM_SHARED((NP_, D), jnp.float32),
        pltpu.SemaphoreType.DMA,
        pltpu.SemaphoreType.DMA,
        pltpu.SemaphoreType.DMA,
        pltpu.SemaphoreType.DMA,
    ],
)(_sc_scat_body)


def _sc_deg_body(col_hbm, ew_hbm, out_hbm, cidx_f, ewv_f, degloc, sem):
    del sem
    cid = lax.axis_index("c")
    sid = lax.axis_index("s")
    wid = sid * NC + cid
    ebase = wid * EPW

    pltpu.sync_copy(col_hbm.at[pl.ds(ebase, EPW)], cidx_f)
    pltpu.sync_copy(ew_hbm.at[pl.ds(ebase, EPW)], ewv_f)

    @pl.loop(0, N // 16)
    def _zero(i):
        degloc[pl.ds(i * 16, 16)] = jnp.zeros((16,), jnp.float32)

    @pl.loop(0, EPW // 16)
    def _upd(e16):
        sl = pl.ds(e16 * 16, 16)
        plsc.addupdate_scatter(degloc, [cidx_f[sl]], ewv_f[sl])

    pltpu.sync_copy(degloc, out_hbm.at[wid])


_sc_deg = functools.partial(
    pl.kernel,
    compiler_params=pltpu.CompilerParams(needs_layout_passes=False),
    out_type=jax.ShapeDtypeStruct((NW, N), jnp.float32),
    mesh=_SC_MESH,
    scratch_types=[
        pltpu.VMEM((EPW,), jnp.int32),
        pltpu.VMEM((EPW,), jnp.float32),
        pltpu.VMEM((N,), jnp.float32),
        pltpu.SemaphoreType.DMA,
    ],
)(_sc_deg_body)


def _dotT(a, b):
    a = a.astype(jnp.bfloat16).astype(jnp.float32)
    b = b.astype(jnp.bfloat16).astype(jnp.float32)
    return lax.dot_general(a, b, (((1,), (1,)), ((), ())),
                           preferred_element_type=jnp.float32)


BE = 8000


def _mlp_body(ea_ref, we1_ref, be1_ref, we2_ref, be2_ref, out_ref):
    h = jnp.maximum(_dotT(ea_ref[...], we1_ref[...]) + be1_ref[...], 0.0)
    out_ref[0, 0, :] = _dotT(h, we2_ref[...])[:, 0] + be2_ref[0]


_mlp = pl.pallas_call(
    _mlp_body,
    grid=(E // BE,),
    in_specs=[
        pl.BlockSpec((BE, DE), lambda i: (i, 0)),
        pl.BlockSpec((H, DE), lambda i: (0, 0)),
        pl.BlockSpec((H,), lambda i: (0,)),
        pl.BlockSpec((1, H), lambda i: (0, 0)),
        pl.BlockSpec((1,), lambda i: (0,)),
    ],
    out_specs=pl.BlockSpec((1, 1, BE), lambda i: (i, 0, 0)),
    out_shape=jax.ShapeDtypeStruct((E // BE, 1, BE), jnp.float32),
)


def _prep_body(degs_ref, x_ref, w1_ref, dis_ref, z_ref):
    deg = jnp.sum(degs_ref[...], axis=0) + 1.0
    dis = jnp.where(deg > 0, lax.rsqrt(jnp.where(deg > 0, deg, 1.0)), 0.0)
    dis_ref[...] = dis
    z_ref[...] = dis[:, None] * _dotT(x_ref[...], w1_ref[...])


_prep = pl.pallas_call(
    _prep_body,
    out_shape=(
        jax.ShapeDtypeStruct((N,), jnp.float32),
        jax.ShapeDtypeStruct((N, D), jnp.float32),
    ),
)


def _bn_relu(out, g, be):
    m = jnp.mean(out, axis=0)
    c = out - m
    v = jnp.mean(c * c, axis=0)
    return jnp.maximum(c * lax.rsqrt(v + 1e-5) * g + be, 0.0)


def _layer_body(s2_ref, z_ref, dis_ref, b_ref, g_ref, be_ref, w_ref, zn_ref):
    dis = dis_ref[...]
    out = dis[:, None] * (s2_ref[0, :N] + s2_ref[1, :N] + z_ref[...]) + b_ref[...]
    xb = _bn_relu(out, g_ref[...], be_ref[...])
    zn_ref[...] = dis[:, None] * _dotT(xb, w_ref[...])


_layer = pl.pallas_call(
    _layer_body,
    out_shape=jax.ShapeDtypeStruct((N, D), jnp.float32),
)


def _final_body(s2_ref, z_ref, dis_ref, b_ref, g_ref, be_ref, batch_ref,
                wr_ref, br_ref, out_ref):
    dis = dis_ref[...]
    out = dis[:, None] * (s2_ref[0, :N] + s2_ref[1, :N] + z_ref[...]) + b_ref[...]
    x3 = _bn_relu(out, g_ref[...], be_ref[...])
    gid = lax.broadcasted_iota(jnp.int32, (N, G), 1)
    oh = (batch_ref[...][:, None] == gid).astype(jnp.float32)
    sums = lax.dot_general(oh.astype(jnp.bfloat16).astype(jnp.float32),
                           x3.astype(jnp.bfloat16).astype(jnp.float32),
                           (((0,), (0,)), ((), ())),
                           preferred_element_type=jnp.float32)
    cnt = jnp.sum(oh, axis=0)
    pooled = sums / jnp.maximum(cnt, 1.0)[:, None]
    out_ref[...] = _dotT(pooled, wr_ref[...])[:, 0] + br_ref[0]


_final = pl.pallas_call(
    _final_body,
    out_shape=jax.ShapeDtypeStruct((G,), jnp.float32),
)


def kernel(x, edge_index, edge_attr, batch,
           W_e1, b_e1, W_e2, b_e2,
           W1, b1, g1, be1, W2, b2, g2, be2, W3, b3, g3, be3, Wr, br):
    row = edge_index[0]
    col = edge_index[1]
    ew = _mlp(edge_attr, W_e1, b_e1, W_e2, b_e2).reshape(E)
    degs = _sc_deg(col, ew)
    dis, z = _prep(degs, x, W1)
    s = _sc_scatter(row, col, ew, z)
    z = _layer(s, z, dis, b1, g1, be1, W2)
    s = _sc_scatter(row, col, ew, z)
    z = _layer(s, z, dis, b2, g2, be2, W3)
    s = _sc_scatter(row, col, ew, z)
    return _final(s, z, dis, b3, g3, be3, batch, Wr, br)


# speedup vs baseline: 1.1899x; 1.1899x over previous
import functools

import jax
import jax.numpy as jnp
from jax import lax
from jax.experimental import pallas as pl
from jax.experimental.pallas import tpu as pltpu
from jax.experimental.pallas import tpu_sc as plsc

N = 10000
E = 320000
D = 128
DE = 16
H = 128
G = 64

NC = 2
NS = 16
NW = NC * NS
EPW = E // NW
K = 80
NCHUNK = EPW // K
NP_ = 10240
RPT = NP_ // NS
ZR = 128

_SC_MESH = plsc.VectorSubcoreMesh(core_axis_name="c", subcore_axis_name="s")


NPAIR = (NCHUNK - 1) // 2


def _sc_scat_body(row_hbm, col_hbm, ew_hbm, z_hbm, out_hbm,
                  ridx_f, cidx_f, cidx_cur, ewv0, ewv1, rows0, rows1, acc,
                  sem0, sem1):
    cid = lax.axis_index("c")
    sid = lax.axis_index("s")
    wid = sid * NC + cid
    ebase = wid * EPW

    pltpu.sync_copy(row_hbm.at[pl.ds(ebase, EPW)], ridx_f)
    pltpu.sync_copy(col_hbm.at[pl.ds(ebase, EPW)], cidx_f)

    @pl.loop(0, K)
    def _zero(i):
        for j in range(D // 16):
            rows0[i, pl.ds(j * 16, 16)] = jnp.zeros((16,), jnp.float32)

    for kk in range(RPT // K):
        pltpu.sync_copy(rows0, acc.at[pl.ds(sid * RPT + kk * K, K)])
    plsc.subcore_barrier()

    def gather_issue(g, rows, sem):
        pltpu.async_copy(z_hbm.at[ridx_f.at[pl.ds(g * K, K)]], rows, sem)

    def gather_wait(g, rows, sem):
        pltpu.make_async_copy(z_hbm.at[ridx_f.at[pl.ds(g * K, K)]], rows,
                              sem).wait()

    def ew_issue(g, buf, sem):
        pltpu.async_copy(ew_hbm.at[pl.ds(ebase + g * K, K)], buf, sem)

    def ew_wait(g, buf, sem):
        pltpu.make_async_copy(ew_hbm.at[pl.ds(ebase + g * K, K)], buf,
                              sem).wait()

    def scale_scatter(g, rows, ewv):
        @pl.loop(0, K // 16)
        def _scale(e16):
            ewvec = ewv[pl.ds(e16 * 16, 16)]
            for l in range(16):
                e = e16 * 16 + l
                s = ewvec[l]
                for j in range(D // 16):
                    sl = pl.ds(j * 16, 16)
                    rows[e, sl] = rows[e, sl] * s

        for i in range(K // 16):
            cidx_cur[pl.ds(i * 16, 16)] = cidx_f[pl.ds(g * K + i * 16, 16)]
        pltpu.sync_copy(rows, acc.at[cidx_cur], add=True)

    gather_issue(0, rows0, sem0)
    ew_issue(0, ewv0, sem0)

    @pl.loop(0, NPAIR)
    def _pair(t):
        g0 = 2 * t
        gather_wait(g0, rows0, sem0)
        ew_wait(g0, ewv0, sem0)
        gather_issue(g0 + 1, rows1, sem1)
        ew_issue(g0 + 1, ewv1, sem1)
        scale_scatter(g0, rows0, ewv0)
        gather_wait(g0 + 1, rows1, sem1)
        ew_wait(g0 + 1, ewv1, sem1)
        gather_issue(g0 + 2, rows0, sem0)
        ew_issue(g0 + 2, ewv0, sem0)
        scale_scatter(g0 + 1, rows1, ewv1)

    gl = NCHUNK - 1
    gather_wait(gl, rows0, sem0)
    ew_wait(gl, ewv0, sem0)
    scale_scatter(gl, rows0, ewv0)

    plsc.subcore_barrier()
    for kk in range(RPT // ZR):
        r0 = sid * RPT + kk * ZR
        pltpu.sync_copy(acc.at[pl.ds(r0, ZR)], out_hbm.at[cid, pl.ds(r0, ZR)])


_sc_scatter = functools.partial(
    pl.kernel,
    out_type=jax.ShapeDtypeStruct((NC, NP_, D), jnp.float32),
    mesh=_SC_MESH,
    scratch_types=[
        pltpu.VMEM((EPW,), jnp.int32),
        pltpu.VMEM((EPW,), jnp.int32),
        pltpu.VMEM((K,), jnp.int32),
        pltpu.VMEM((K,), jnp.float32),
        pltpu.VMEM((K,), jnp.float32),
        pltpu.VMEM((K, D), jnp.float32),
        pltpu.VMEM((K, D), jnp.float32),
        pltpu.VMEM_SHARED((NP_, D), jnp.float32),
        pltpu.SemaphoreType.DMA,
        pltpu.SemaphoreType.DMA,
    ],
)(_sc_scat_body)


def _sc_deg_body(col_hbm, ew_hbm, out_hbm, cidx_f, ewv_f, degloc, sem):
    del sem
    cid = lax.axis_index("c")
    sid = lax.axis_index("s")
    wid = sid * NC + cid
    ebase = wid * EPW

    pltpu.sync_copy(col_hbm.at[pl.ds(ebase, EPW)], cidx_f)
    pltpu.sync_copy(ew_hbm.at[pl.ds(ebase, EPW)], ewv_f)

    @pl.loop(0, N // 16)
    def _zero(i):
        degloc[pl.ds(i * 16, 16)] = jnp.zeros((16,), jnp.float32)

    @pl.loop(0, EPW // 16)
    def _upd(e16):
        sl = pl.ds(e16 * 16, 16)
        plsc.addupdate_scatter(degloc, [cidx_f[sl]], ewv_f[sl])

    pltpu.sync_copy(degloc, out_hbm.at[wid])


_sc_deg = functools.partial(
    pl.kernel,
    compiler_params=pltpu.CompilerParams(needs_layout_passes=False),
    out_type=jax.ShapeDtypeStruct((NW, N), jnp.float32),
    mesh=_SC_MESH,
    scratch_types=[
        pltpu.VMEM((EPW,), jnp.int32),
        pltpu.VMEM((EPW,), jnp.float32),
        pltpu.VMEM((N,), jnp.float32),
        pltpu.SemaphoreType.DMA,
    ],
)(_sc_deg_body)


def _dotT(a, b):
    a = a.astype(jnp.bfloat16).astype(jnp.float32)
    b = b.astype(jnp.bfloat16).astype(jnp.float32)
    return lax.dot_general(a, b, (((1,), (1,)), ((), ())),
                           preferred_element_type=jnp.float32)


BE = 8000


def _mlp_body(ea_ref, we1_ref, be1_ref, we2_ref, be2_ref, out_ref):
    h = jnp.maximum(_dotT(ea_ref[...], we1_ref[...]) + be1_ref[...], 0.0)
    out_ref[0, 0, :] = _dotT(h, we2_ref[...])[:, 0] + be2_ref[0]


_mlp = pl.pallas_call(
    _mlp_body,
    grid=(E // BE,),
    in_specs=[
        pl.BlockSpec((BE, DE), lambda i: (i, 0)),
        pl.BlockSpec((H, DE), lambda i: (0, 0)),
        pl.BlockSpec((H,), lambda i: (0,)),
        pl.BlockSpec((1, H), lambda i: (0, 0)),
        pl.BlockSpec((1,), lambda i: (0,)),
    ],
    out_specs=pl.BlockSpec((1, 1, BE), lambda i: (i, 0, 0)),
    out_shape=jax.ShapeDtypeStruct((E // BE, 1, BE), jnp.float32),
)


def _prep_body(degs_ref, x_ref, w1_ref, dis_ref, z_ref):
    deg = jnp.sum(degs_ref[...], axis=0) + 1.0
    dis = jnp.where(deg > 0, lax.rsqrt(jnp.where(deg > 0, deg, 1.0)), 0.0)
    dis_ref[...] = dis
    z_ref[...] = dis[:, None] * _dotT(x_ref[...], w1_ref[...])


_prep = pl.pallas_call(
    _prep_body,
    out_shape=(
        jax.ShapeDtypeStruct((N,), jnp.float32),
        jax.ShapeDtypeStruct((N, D), jnp.float32),
    ),
)


def _bn_relu(out, g, be):
    m = jnp.mean(out, axis=0)
    c = out - m
    v = jnp.mean(c * c, axis=0)
    return jnp.maximum(c * lax.rsqrt(v + 1e-5) * g + be, 0.0)


def _layer_body(s2_ref, z_ref, dis_ref, b_ref, g_ref, be_ref, w_ref, zn_ref):
    dis = dis_ref[...]
    out = dis[:, None] * (s2_ref[0, :N] + s2_ref[1, :N] + z_ref[...]) + b_ref[...]
    xb = _bn_relu(out, g_ref[...], be_ref[...])
    zn_ref[...] = dis[:, None] * _dotT(xb, w_ref[...])


_layer = pl.pallas_call(
    _layer_body,
    out_shape=jax.ShapeDtypeStruct((N, D), jnp.float32),
)


def _final_body(s2_ref, z_ref, dis_ref, b_ref, g_ref, be_ref, batch_ref,
                wr_ref, br_ref, out_ref):
    dis = dis_ref[...]
    out = dis[:, None] * (s2_ref[0, :N] + s2_ref[1, :N] + z_ref[...]) + b_ref[...]
    x3 = _bn_relu(out, g_ref[...], be_ref[...])
    gid = lax.broadcasted_iota(jnp.int32, (N, G), 1)
    oh = (batch_ref[...][:, None] == gid).astype(jnp.float32)
    sums = lax.dot_general(oh.astype(jnp.bfloat16).astype(jnp.float32),
                           x3.astype(jnp.bfloat16).astype(jnp.float32),
                           (((0,), (0,)), ((), ())),
                           preferred_element_type=jnp.float32)
    cnt = jnp.sum(oh, axis=0)
    pooled = sums / jnp.maximum(cnt, 1.0)[:, None]
    out_ref[...] = _dotT(pooled, wr_ref[...])[:, 0] + br_ref[0]


_final = pl.pallas_call(
    _final_body,
    out_shape=jax.ShapeDtypeStruct((G,), jnp.float32),
)


def kernel(x, edge_index, edge_attr, batch,
           W_e1, b_e1, W_e2, b_e2,
           W1, b1, g1, be1, W2, b2, g2, be2, W3, b3, g3, be3, Wr, br):
    row = edge_index[0]
    col = edge_index[1]
    ew = _mlp(edge_attr, W_e1, b_e1, W_e2, b_e2).reshape(E)
    degs = _sc_deg(col, ew)
    dis, z = _prep(degs, x, W1)
    s = _sc_scatter(row, col, ew, z)
    z = _layer(s, z, dis, b1, g1, be1, W2)
    s = _sc_scatter(row, col, ew, z)
    z = _layer(s, z, dis, b2, g2, be2, W3)
    s = _sc_scatter(row, col, ew, z)
    return _final(s, z, dis, b3, g3, be3, batch, Wr, br)
